# baseline (speedup 1.0000x reference)
import functools

import jax
import jax.numpy as jnp
from jax import lax
from jax.experimental import pallas as pl
from jax.experimental.pallas import tpu as pltpu
from jax.experimental.pallas import tpu_sc as plsc

B, T, N_EMBD = 4, 2048, 768
NUM_EXPERTS, TOP_K = 8, 2
CITY_DIM = 32
D_FF = 4 * N_EMBD
N_TOK = B * T
N_SLOT = N_TOK * TOP_K

BLK = 256
NBLK = 72
PADN = NBLK * BLK

_NC, _NS = 2, 16
_NW = _NC * _NS
_SLOT_PER_W = N_SLOT // _NW
_TOK_PER_W = _SLOT_PER_W

_SQRT_2_OVER_PI = 0.7978845608028654


def _gelu_tanh(x):
    return 0.5 * x * (1.0 + jnp.tanh(_SQRT_2_OVER_PI * (x + 0.044715 * x * x * x)))


def _router_body(x_ref, d1_ref, d2_ref, d3_ref, d4_ref,
                 wx_ref, w1_ref, w2_ref, w3_ref, w4_ref, bias_ref,
                 gate1_ref, i1_ref, i2_ref, w1o_ref, w2o_ref):
    logits = jnp.dot(x_ref[...], wx_ref[...], preferred_element_type=jnp.float32)
    logits += jnp.dot(d1_ref[...], w1_ref[...], preferred_element_type=jnp.float32)
    logits += jnp.dot(d2_ref[...], w2_ref[...], preferred_element_type=jnp.float32)
    logits += jnp.dot(d3_ref[...], w3_ref[...], preferred_element_type=jnp.float32)
    logits += jnp.dot(d4_ref[...], w4_ref[...], preferred_element_type=jnp.float32)
    logits += bias_ref[...]

    m1 = jnp.max(logits, axis=-1, keepdims=True)
    e = jnp.exp(logits - m1)
    gate1_ref[...] = e / jnp.sum(e, axis=-1, keepdims=True)

    iota = jax.lax.broadcasted_iota(jnp.int32, logits.shape, 1)
    i1 = jnp.min(jnp.where(logits == m1, iota, NUM_EXPERTS), axis=-1,
                 keepdims=True)
    masked = jnp.where(iota == i1, -jnp.inf, logits)
    m2 = jnp.max(masked, axis=-1, keepdims=True)
    i2 = jnp.min(jnp.where(masked == m2, iota, NUM_EXPERTS), axis=-1,
                 keepdims=True)
    w_top1 = 1.0 / (1.0 + jnp.exp(m2 - m1))
    i1_ref[...] = i1
    i2_ref[...] = i2
    w1o_ref[...] = w_top1
    w2o_ref[...] = 1.0 - w_top1


def _run_router(x2d, d1, d2, d3, d4, router_w, bias_full):
    blk = 1024
    grid = (N_TOK // blk,)
    wx = router_w[:N_EMBD]
    o = N_EMBD + CITY_DIM
    w1 = router_w[o:o + 192]
    w2 = router_w[o + 192:o + 384]
    w3 = router_w[o + 384:o + 480]
    w4 = router_w[o + 480:o + 576]

    def tok_block(i):
        return (i, 0)

    def full(i):
        return (0, 0)

    col = pl.BlockSpec((blk, 1), tok_block)
    return pl.pallas_call(
        _router_body,
        grid=grid,
        in_specs=[
            pl.BlockSpec((blk, N_EMBD), tok_block),
            pl.BlockSpec((blk, 192), tok_block),
            pl.BlockSpec((blk, 192), tok_block),
            pl.BlockSpec((blk, 96), tok_block),
            pl.BlockSpec((blk, 96), tok_block),
            pl.BlockSpec((N_EMBD, NUM_EXPERTS), full),
            pl.BlockSpec((192, NUM_EXPERTS), full),
            pl.BlockSpec((192, NUM_EXPERTS), full),
            pl.BlockSpec((96, NUM_EXPERTS), full),
            pl.BlockSpec((96, NUM_EXPERTS), full),
            pl.BlockSpec((1, NUM_EXPERTS), full),
        ],
        out_specs=[
            pl.BlockSpec((blk, NUM_EXPERTS), tok_block),
            col, col, col, col,
        ],
        out_shape=[
            jax.ShapeDtypeStruct((N_TOK, NUM_EXPERTS), jnp.float32),
            jax.ShapeDtypeStruct((N_TOK, 1), jnp.int32),
            jax.ShapeDtypeStruct((N_TOK, 1), jnp.int32),
            jax.ShapeDtypeStruct((N_TOK, 1), jnp.float32),
            jax.ShapeDtypeStruct((N_TOK, 1), jnp.float32),
        ],
    )(x2d, d1, d2, d3, d4, wx, w1, w2, w3, w4, bias_full)


_GRP = 16
_CHUNK = 128
_N_CHUNK = _SLOT_PER_W // _CHUNK
_POS_GRPS = _SLOT_PER_W // _GRP


def _sc_count_body(ex0_hbm, ex1_hbm, cnt_hbm, ex_v, cnt_stage):
    c = lax.axis_index("c")
    s = lax.axis_index("s")
    w = 2 * s + c
    lane = lax.iota(jnp.int32, 16)

    @pl.when(w < 16)
    def _():
        pltpu.sync_copy(ex0_hbm.at[pl.ds(512 * w, 512)],
                        ex_v.at[pl.ds(0, 512)])

    @pl.when(w >= 16)
    def _():
        pltpu.sync_copy(ex1_hbm.at[pl.ds(512 * (w - 16), 512)],
                        ex_v.at[pl.ds(0, 512)])
    zero16 = jnp.zeros((16,), jnp.int32)

    def body(g, a):
        v = ex_v[pl.ds(g * 16, 16)]
        for e in range(NUM_EXPERTS):
            p = jnp.sum((v == e).astype(jnp.int32))
            a = a + jnp.where(lane == e, p, 0)
        return a

    cnt = lax.fori_loop(0, 512 // 16, body, zero16)
    cnt_stage[0, :] = cnt
    pltpu.sync_copy(cnt_stage, cnt_hbm.at[pl.ds(w, 1)])


def _run_sc_count(ex0, ex1):
    mesh = plsc.VectorSubcoreMesh(core_axis_name="c", subcore_axis_name="s")
    f = pl.kernel(
        _sc_count_body,
        out_type=jax.ShapeDtypeStruct((_NW, 16), jnp.int32),
        mesh=mesh,
        scratch_types=[
            pltpu.VMEM((1024,), jnp.int32),
            pltpu.VMEM((1, 16), jnp.int32),
        ],
        compiler_params=pltpu.CompilerParams(needs_layout_passes=False),
    )
    return f(ex0, ex1)


def _sc_route_body(ex0_hbm, ex1_hbm, x_hbm, cnt_hbm,
                   pos_hbm, be_hbm, px_hbm,
                   ex_v, all_v, pos_v, be_v, xbuf, xbuf2, xbuf3, xbuf4,
                   seml0, seml1, seml2, seml3, sems0, sems1, sems2, sems3):
    c = lax.axis_index("c")
    s = lax.axis_index("s")
    w = 2 * s + c
    lane = lax.iota(jnp.int32, 16)
    zero16 = jnp.zeros((16,), jnp.int32)

    pltpu.sync_copy(cnt_hbm, all_v)
    tot = zero16
    pre = zero16
    for r in range(_NW):
        row = all_v[r, :]
        tot = tot + row
        pre = pre + row * (r < w).astype(jnp.int32)
    padded = ((tot + (BLK - 1)) // BLK) * BLK
    base_excl = plsc.cumsum(padded) - padded
    start = base_excl + pre

    @pl.when(w < 16)
    def _():
        pltpu.sync_copy(ex0_hbm.at[pl.ds(512 * w, 512)],
                        ex_v.at[pl.ds(0, 512)])

    @pl.when(w >= 16)
    def _():
        pltpu.sync_copy(ex1_hbm.at[pl.ds(512 * (w - 16), 512)],
                        ex_v.at[pl.ds(0, 512)])

    def pos_grp(g, start_vec):
        v = ex_v[pl.ds(g * 16, 16)]
        posv = jnp.zeros((16,), jnp.int32)
        upd = start_vec
        for e in range(NUM_EXPERTS):
            mi = (v == e).astype(jnp.int32)
            csum = plsc.cumsum(mi)
            start_e = jnp.sum(jnp.where(lane == e, start_vec, 0))
            posv = posv + mi * (start_e + csum - 1)
            cnt_e = jnp.sum(mi)
            upd = upd + jnp.where(lane == e, cnt_e, 0)
        pos_v[g // 2, pl.ds((g % 2) * 16, 16)] = posv
        return upd

    lax.fori_loop(0, _POS_GRPS, pos_grp, start)

    pltpu.sync_copy(pos_v, pos_hbm.at[pl.ds(16 * w, 16)])

    @pl.when(w == 0)
    def _():
        base_blk = base_excl // BLK
        sb = [jnp.sum(jnp.where(lane == e, base_blk, 0))
              for e in range(NUM_EXPERTS)]
        for j in range(128 // 16):
            blkid = lane + 16 * j
            bev = jnp.zeros((16,), jnp.int32)
            for e in range(NUM_EXPERTS):
                bev = bev + (blkid >= sb[e]).astype(jnp.int32)
            be_v[pl.ds(16 * j, 16)] = bev - 1
        pltpu.sync_copy(be_v, be_hbm)

    tok_base = 512 * (w % 16)
    xbs = (xbuf, xbuf2, xbuf3, xbuf4)
    sls = (seml0, seml1, seml2, seml3)
    sss = (sems0, sems1, sems2, sems3)

    def ld(ch):
        pltpu.async_copy(x_hbm.at[pl.ds(tok_base + 32 * ch, 32)],
                         xbs[ch % 4], sls[ch % 4])

    def idx_of(ch):
        return pos_v.at[ch]

    ld(0)
    for ch in range(16):
        p = ch % 4
        if ch + 1 < 16:
            q = (ch + 1) % 4
            if ch + 1 >= 4:
                pltpu.make_async_copy(xbs[q], px_hbm.at[idx_of(ch - 3)],
                                      sss[q]).wait()
            ld(ch + 1)
        pltpu.make_async_copy(x_hbm.at[pl.ds(tok_base + 32 * ch, 32)],
                              xbs[p], sls[p]).wait()
        pltpu.async_copy(xbs[p], px_hbm.at[idx_of(ch)], sss[p])
    for ch in range(12, 16):
        p = ch % 4
        pltpu.make_async_copy(xbs[p], px_hbm.at[idx_of(ch)], sss[p]).wait()


def _run_sc_route(ex0, ex1, x2d, cnt):
    mesh = plsc.VectorSubcoreMesh(core_axis_name="c", subcore_axis_name="s")
    f = pl.kernel(
        _sc_route_body,
        out_type=[
            jax.ShapeDtypeStruct((N_SLOT // 32, 32), jnp.int32),
            jax.ShapeDtypeStruct((128,), jnp.int32),
            jax.ShapeDtypeStruct((PADN, N_EMBD), jnp.float32),
        ],
        mesh=mesh,
        scratch_types=[
            pltpu.VMEM((1024,), jnp.int32),
            pltpu.VMEM((_NW, 16), jnp.int32),
            pltpu.VMEM((16, 32), jnp.int32),
            pltpu.VMEM((128,), jnp.int32),
            pltpu.VMEM((32, N_EMBD), jnp.float32),
            pltpu.VMEM((32, N_EMBD), jnp.float32),
            pltpu.VMEM((32, N_EMBD), jnp.float32),
            pltpu.VMEM((32, N_EMBD), jnp.float32),
            pltpu.SemaphoreType.DMA,
            pltpu.SemaphoreType.DMA,
            pltpu.SemaphoreType.DMA,
            pltpu.SemaphoreType.DMA,
            pltpu.SemaphoreType.DMA,
            pltpu.SemaphoreType.DMA,
            pltpu.SemaphoreType.DMA,
            pltpu.SemaphoreType.DMA,
        ],
        compiler_params=pltpu.CompilerParams(needs_layout_passes=False),
    )
    return f(ex0, ex1, x2d, cnt)


def _gffn_body(be_ref, x_ref, fcw_ref, fcb_ref, pjw_ref, pjb_ref,
               out_ref):
    xb = x_ref[...].astype(jnp.bfloat16)
    hid = jnp.dot(xb, fcw_ref[0], preferred_element_type=jnp.float32)
    hid = _gelu_tanh(hid + fcb_ref[0])
    out = jnp.dot(hid.astype(jnp.bfloat16), pjw_ref[0],
                  preferred_element_type=jnp.float32)
    out_ref[...] = out + pjb_ref[0]


def _run_grouped_ffn(be, perm_x, fc_w, fc_b, proj_w, proj_b):
    grid_spec = pltpu.PrefetchScalarGridSpec(
        num_scalar_prefetch=1,
        grid=(NBLK,),
        in_specs=[
            pl.BlockSpec((BLK, N_EMBD), lambda g, be: (g, 0)),
            pl.BlockSpec((1, N_EMBD, D_FF), lambda g, be: (be[g], 0, 0)),
            pl.BlockSpec((1, 1, D_FF), lambda g, be: (be[g], 0, 0)),
            pl.BlockSpec((1, D_FF, N_EMBD), lambda g, be: (be[g], 0, 0)),
            pl.BlockSpec((1, 1, N_EMBD), lambda g, be: (be[g], 0, 0)),
        ],
        out_specs=pl.BlockSpec((BLK, N_EMBD), lambda g, be: (g, 0)),
    )
    return pl.pallas_call(
        _gffn_body,
        grid_spec=grid_spec,
        out_shape=jax.ShapeDtypeStruct((PADN, N_EMBD), jnp.float32),
    )(be, perm_x,
      fc_w.astype(jnp.bfloat16),
      fc_b.reshape(NUM_EXPERTS, 1, D_FF),
      proj_w.astype(jnp.bfloat16),
      proj_b.reshape(NUM_EXPERTS, 1, N_EMBD))


_CTOK = 16


_N_CCH = 256 // _CTOK


def _sc_combine_body(yw_hbm, pos_hbm, gw0_hbm, gw1_hbm, out_hbm,
                     idx_v, gw0_v, gw1_v, g0a, g1a, g0b, g1b, st,
                     semga, semgb, sems):
    c = lax.axis_index("c")
    s = lax.axis_index("s")
    w = 2 * s + c
    tok_base = _TOK_PER_W // 2 * w

    pairs = ((g0a, g1a, semga), (g0b, g1b, semgb))

    pltpu.sync_copy(pos_hbm.at[pl.ds(8 * w, 8)], idx_v.at[pl.ds(0, 8)])
    pltpu.sync_copy(pos_hbm.at[pl.ds(256 + 8 * w, 8)], idx_v.at[pl.ds(8, 8)])
    pltpu.sync_copy(gw0_hbm.at[pl.ds(tok_base, 256)], gw0_v.at[pl.ds(0, 256)])
    pltpu.sync_copy(gw1_hbm.at[pl.ds(tok_base, 256)], gw1_v.at[pl.ds(0, 256)])

    def idx0(ch):
        return idx_v.at[ch // 2, pl.ds((ch % 2) * 16, 16)]

    def idx1(ch):
        return idx_v.at[8 + ch // 2, pl.ds((ch % 2) * 16, 16)]

    def start_gather(ch):
        g0, g1, sg = pairs[ch % 2]
        pltpu.async_copy(yw_hbm.at[idx0(ch)], g0, sg)
        pltpu.async_copy(yw_hbm.at[idx1(ch)], g1, sg)

    start_gather(0)
    start_gather(1)
    for ch in range(_N_CCH):
        g0, g1, sg = pairs[ch % 2]
        pltpu.make_async_copy(yw_hbm.at[idx0(ch)], g0, sg).wait()
        pltpu.make_async_copy(yw_hbm.at[idx1(ch)], g1, sg).wait()
        if ch >= 1:
            tp = tok_base + _CTOK * (ch - 1)
            pltpu.make_async_copy(st, out_hbm.at[pl.ds(tp, _CTOK)],
                                  sems).wait()

        def add_row(r, _):
            gv0 = jnp.full((16,), gw0_v[pl.ds(_CTOK * ch + r, 16)][0],
                           jnp.float32)
            gv1 = jnp.full((16,), gw1_v[pl.ds(_CTOK * ch + r, 16)][0],
                           jnp.float32)
            for cc in range(N_EMBD // 16):
                st[r, pl.ds(cc * 16, 16)] = (g0[r, pl.ds(cc * 16, 16)] * gv0
                                             + g1[r, pl.ds(cc * 16, 16)] * gv1)
            return 0

        lax.fori_loop(0, _CTOK, add_row, 0)
        t0 = tok_base + _CTOK * ch
        pltpu.async_copy(st, out_hbm.at[pl.ds(t0, _CTOK)], sems)
        if ch + 2 < _N_CCH:
            start_gather(ch + 2)
    t0 = tok_base + _CTOK * (_N_CCH - 1)
    pltpu.make_async_copy(st, out_hbm.at[pl.ds(t0, _CTOK)], sems).wait()


def _run_sc_combine(yw, pos, gw0, gw1):
    mesh = plsc.VectorSubcoreMesh(core_axis_name="c", subcore_axis_name="s")
    buf = pltpu.VMEM((_CTOK, N_EMBD), jnp.float32)
    f = pl.kernel(
        _sc_combine_body,
        out_type=jax.ShapeDtypeStruct((N_TOK, N_EMBD), jnp.float32),
        mesh=mesh,
        scratch_types=[
            pltpu.VMEM((16, 32), jnp.int32),
            pltpu.VMEM((272,), jnp.float32),
            pltpu.VMEM((272,), jnp.float32),
            buf, buf, buf, buf, buf,
            pltpu.SemaphoreType.DMA,
            pltpu.SemaphoreType.DMA,
            pltpu.SemaphoreType.DMA,
        ],
        compiler_params=pltpu.CompilerParams(needs_layout_passes=False),
    )
    return f(yw, pos, gw0.reshape(-1), gw1.reshape(-1))


def kernel(x, city, delta_t_info, delta_dis_info, delta_rg_info,
           delta_entropy_info, city_embeddings, router_w, router_b,
           fc_w, fc_b, proj_w, proj_b):
    b, t, d = x.shape
    n = b * t
    x2d = x.reshape(n, d)
    d1 = delta_t_info.reshape(n, -1)
    d2 = delta_dis_info.reshape(n, -1)
    d3 = delta_rg_info.reshape(n, -1)
    d4 = delta_entropy_info.reshape(n, -1)

    ce = city_embeddings[city]
    w_ce = jax.lax.dynamic_slice_in_dim(router_w, N_EMBD, CITY_DIM, 0)
    bias_full = (jnp.dot(ce, w_ce) + router_b).reshape(1, NUM_EXPERTS)

    gate1, i1, i2, w1, w2 = _run_router(x2d, d1, d2, d3, d4, router_w,
                                        bias_full)
    ex0 = i1.reshape(-1)
    ex1 = i2.reshape(-1)

    cnt = _run_sc_count(ex0, ex1)
    pos, be_pad, perm_x = _run_sc_route(ex0, ex1, x2d, cnt)
    yw = _run_grouped_ffn(be_pad[:NBLK], perm_x, fc_w, fc_b, proj_w, proj_b)
    out2d = _run_sc_combine(yw, pos, w1, w2)
    return out2d.reshape(b, t, d), gate1.reshape(b, t, NUM_EXPERTS)


# speedup vs baseline: 1.0238x; 1.0238x over previous
import functools

import jax
import jax.numpy as jnp
from jax import lax
from jax.experimental import pallas as pl
from jax.experimental.pallas import tpu as pltpu
from jax.experimental.pallas import tpu_sc as plsc

B, T, N_EMBD = 4, 2048, 768
NUM_EXPERTS, TOP_K = 8, 2
CITY_DIM = 32
D_FF = 4 * N_EMBD
N_TOK = B * T
N_SLOT = N_TOK * TOP_K

BLK = 256
NBLK = 72
PADN = NBLK * BLK

_NC, _NS = 2, 16
_NW = _NC * _NS
_SLOT_PER_W = N_SLOT // _NW
_TOK_PER_W = _SLOT_PER_W

_SQRT_2_OVER_PI = 0.7978845608028654


def _gelu_tanh(x):
    return 0.5 * x * (1.0 + jnp.tanh(_SQRT_2_OVER_PI * (x + 0.044715 * x * x * x)))


def _router_body(x_ref, d1_ref, d2_ref, d3_ref, d4_ref,
                 wx_ref, w1_ref, w2_ref, w3_ref, w4_ref, bias_ref,
                 gate1_ref, i1_ref, i2_ref, w1o_ref, w2o_ref, cnt_ref):
    logits = jnp.dot(x_ref[...], wx_ref[...], preferred_element_type=jnp.float32)
    logits += jnp.dot(d1_ref[...], w1_ref[...], preferred_element_type=jnp.float32)
    logits += jnp.dot(d2_ref[...], w2_ref[...], preferred_element_type=jnp.float32)
    logits += jnp.dot(d3_ref[...], w3_ref[...], preferred_element_type=jnp.float32)
    logits += jnp.dot(d4_ref[...], w4_ref[...], preferred_element_type=jnp.float32)
    logits += bias_ref[...]

    m1 = jnp.max(logits, axis=-1, keepdims=True)
    e = jnp.exp(logits - m1)
    gate1_ref[...] = e / jnp.sum(e, axis=-1, keepdims=True)

    iota = jax.lax.broadcasted_iota(jnp.int32, logits.shape, 1)
    i1 = jnp.min(jnp.where(logits == m1, iota, NUM_EXPERTS), axis=-1,
                 keepdims=True)
    masked = jnp.where(iota == i1, -jnp.inf, logits)
    m2 = jnp.max(masked, axis=-1, keepdims=True)
    i2 = jnp.min(jnp.where(masked == m2, iota, NUM_EXPERTS), axis=-1,
                 keepdims=True)
    w_top1 = 1.0 / (1.0 + jnp.exp(m2 - m1))
    i1_ref[...] = i1
    i2_ref[...] = i2
    w1o_ref[...] = w_top1
    w2o_ref[...] = 1.0 - w_top1

    lane16 = jax.lax.broadcasted_iota(jnp.int32, (512, 16), 1)
    rows = []
    for src_col, _k in ((i1, 0), (i2, 1)):
        for h in range(2):
            sl = src_col[h * 512:(h + 1) * 512]
            oh = (sl == lane16).astype(jnp.int32)
            rows.append(jnp.sum(oh, axis=0, keepdims=True))
    cnt_ref[...] = jnp.concatenate(rows, axis=0)[None]


def _run_router(x2d, d1, d2, d3, d4, router_w, bias_full):
    blk = 1024
    grid = (N_TOK // blk,)
    wx = router_w[:N_EMBD]
    o = N_EMBD + CITY_DIM
    w1 = router_w[o:o + 192]
    w2 = router_w[o + 192:o + 384]
    w3 = router_w[o + 384:o + 480]
    w4 = router_w[o + 480:o + 576]

    def tok_block(i):
        return (i, 0)

    def full(i):
        return (0, 0)

    col = pl.BlockSpec((blk, 1), tok_block)
    return pl.pallas_call(
        _router_body,
        grid=grid,
        in_specs=[
            pl.BlockSpec((blk, N_EMBD), tok_block),
            pl.BlockSpec((blk, 192), tok_block),
            pl.BlockSpec((blk, 192), tok_block),
            pl.BlockSpec((blk, 96), tok_block),
            pl.BlockSpec((blk, 96), tok_block),
            pl.BlockSpec((N_EMBD, NUM_EXPERTS), full),
            pl.BlockSpec((192, NUM_EXPERTS), full),
            pl.BlockSpec((192, NUM_EXPERTS), full),
            pl.BlockSpec((96, NUM_EXPERTS), full),
            pl.BlockSpec((96, NUM_EXPERTS), full),
            pl.BlockSpec((1, NUM_EXPERTS), full),
        ],
        out_specs=[
            pl.BlockSpec((blk, NUM_EXPERTS), tok_block),
            col, col, col, col,
            pl.BlockSpec((1, 4, 16), lambda i: (i, 0, 0)),
        ],
        out_shape=[
            jax.ShapeDtypeStruct((N_TOK, NUM_EXPERTS), jnp.float32),
            jax.ShapeDtypeStruct((N_TOK, 1), jnp.int32),
            jax.ShapeDtypeStruct((N_TOK, 1), jnp.int32),
            jax.ShapeDtypeStruct((N_TOK, 1), jnp.float32),
            jax.ShapeDtypeStruct((N_TOK, 1), jnp.float32),
            jax.ShapeDtypeStruct((_NW // 4, 4, 16), jnp.int32),
        ],
    )(x2d, d1, d2, d3, d4, wx, w1, w2, w3, w4, bias_full)


_GRP = 16
_CHUNK = 128
_N_CHUNK = _SLOT_PER_W // _CHUNK
_POS_GRPS = _SLOT_PER_W // _GRP


def _sc_route_body(ex0_hbm, ex1_hbm, x_hbm, cnt_hbm,
                   pos_hbm, be_hbm, px_hbm,
                   ex_v, all_v, pos_v, be_v, xbuf, xbuf2, xbuf3, xbuf4,
                   seml0, seml1, seml2, seml3, sems0, sems1, sems2, sems3):
    c = lax.axis_index("c")
    s = lax.axis_index("s")
    w = 2 * s + c
    lane = lax.iota(jnp.int32, 16)
    zero16 = jnp.zeros((16,), jnp.int32)

    pltpu.sync_copy(cnt_hbm, all_v)
    row_w = 4 * ((w % 16) // 2) + 2 * (w // 16) + (w % 16) % 2
    tot = zero16
    pre = zero16
    for r in range(_NW):
        row = all_v[r // 4, r % 4, :]
        tot = tot + row
        pre = pre + row * (r < row_w).astype(jnp.int32)
    padded = ((tot + (BLK - 1)) // BLK) * BLK
    base_excl = plsc.cumsum(padded) - padded
    start = base_excl + pre

    @pl.when(w < 16)
    def _():
        pltpu.sync_copy(ex0_hbm.at[pl.ds(512 * w, 512)],
                        ex_v.at[pl.ds(0, 512)])

    @pl.when(w >= 16)
    def _():
        pltpu.sync_copy(ex1_hbm.at[pl.ds(512 * (w - 16), 512)],
                        ex_v.at[pl.ds(0, 512)])

    def pos_grp(g, start_vec):
        v = ex_v[pl.ds(g * 16, 16)]
        posv = jnp.zeros((16,), jnp.int32)
        upd = start_vec
        for e in range(NUM_EXPERTS):
            mi = (v == e).astype(jnp.int32)
            csum = plsc.cumsum(mi)
            start_e = jnp.sum(jnp.where(lane == e, start_vec, 0))
            posv = posv + mi * (start_e + csum - 1)
            cnt_e = jnp.sum(mi)
            upd = upd + jnp.where(lane == e, cnt_e, 0)
        pos_v[g // 2, pl.ds((g % 2) * 16, 16)] = posv
        return upd

    lax.fori_loop(0, _POS_GRPS, pos_grp, start)

    pltpu.sync_copy(pos_v, pos_hbm.at[pl.ds(16 * w, 16)])

    @pl.when(w == 0)
    def _():
        base_blk = base_excl // BLK
        tot_blk = jnp.sum(padded // BLK)
        sb = [jnp.sum(jnp.where(lane == e, base_blk, 0))
              for e in range(NUM_EXPERTS)]
        for j in range(128 // 16):
            blkid = lane + 16 * j
            bev = jnp.zeros((16,), jnp.int32)
            for e in range(NUM_EXPERTS):
                bev = bev + (blkid >= sb[e]).astype(jnp.int32)
            be_v[0, pl.ds(16 * j, 16)] = bev - 1
            be_v[1, pl.ds(16 * j, 16)] = (blkid < tot_blk).astype(jnp.int32)
        pltpu.sync_copy(be_v, be_hbm)

    tok_base = 512 * (w % 16)
    xbs = (xbuf, xbuf2, xbuf3, xbuf4)
    sls = (seml0, seml1, seml2, seml3)
    sss = (sems0, sems1, sems2, sems3)

    def ld(ch):
        pltpu.async_copy(x_hbm.at[pl.ds(tok_base + 32 * ch, 32)],
                         xbs[ch % 4], sls[ch % 4])

    def idx_of(ch):
        return pos_v.at[ch]

    ld(0)
    for ch in range(16):
        p = ch % 4
        if ch + 1 < 16:
            q = (ch + 1) % 4
            if ch + 1 >= 4:
                pltpu.make_async_copy(xbs[q], px_hbm.at[idx_of(ch - 3)],
                                      sss[q]).wait()
            ld(ch + 1)
        pltpu.make_async_copy(x_hbm.at[pl.ds(tok_base + 32 * ch, 32)],
                              xbs[p], sls[p]).wait()
        pltpu.async_copy(xbs[p], px_hbm.at[idx_of(ch)], sss[p])
    for ch in range(12, 16):
        p = ch % 4
        pltpu.make_async_copy(xbs[p], px_hbm.at[idx_of(ch)], sss[p]).wait()


def _run_sc_route(ex0, ex1, x2d, cnt):
    mesh = plsc.VectorSubcoreMesh(core_axis_name="c", subcore_axis_name="s")
    f = pl.kernel(
        _sc_route_body,
        out_type=[
            jax.ShapeDtypeStruct((N_SLOT // 32, 32), jnp.int32),
            jax.ShapeDtypeStruct((2, 128), jnp.int32),
            jax.ShapeDtypeStruct((PADN, N_EMBD), jnp.float32),
        ],
        mesh=mesh,
        scratch_types=[
            pltpu.VMEM((1024,), jnp.int32),
            pltpu.VMEM((_NW // 4, 4, 16), jnp.int32),
            pltpu.VMEM((16, 32), jnp.int32),
            pltpu.VMEM((2, 128), jnp.int32),
            pltpu.VMEM((32, N_EMBD), jnp.float32),
            pltpu.VMEM((32, N_EMBD), jnp.float32),
            pltpu.VMEM((32, N_EMBD), jnp.float32),
            pltpu.VMEM((32, N_EMBD), jnp.float32),
            pltpu.SemaphoreType.DMA,
            pltpu.SemaphoreType.DMA,
            pltpu.SemaphoreType.DMA,
            pltpu.SemaphoreType.DMA,
            pltpu.SemaphoreType.DMA,
            pltpu.SemaphoreType.DMA,
            pltpu.SemaphoreType.DMA,
            pltpu.SemaphoreType.DMA,
        ],
        compiler_params=pltpu.CompilerParams(needs_layout_passes=False),
    )
    return f(ex0, ex1, x2d, cnt)


def _gffn_body(be_ref, x_ref, fcw_ref, fcb_ref, pjw_ref, pjb_ref,
               out_ref):
    g = pl.program_id(0)

    @pl.when(be_ref[1, g] == 1)
    def _():
        xb = x_ref[...].astype(jnp.bfloat16)
        hid = jnp.dot(xb, fcw_ref[0], preferred_element_type=jnp.float32)
        hid = _gelu_tanh(hid + fcb_ref[0])
        out = jnp.dot(hid.astype(jnp.bfloat16), pjw_ref[0],
                      preferred_element_type=jnp.float32)
        out_ref[...] = out + pjb_ref[0]


def _run_grouped_ffn(be, perm_x, fc_w, fc_b, proj_w, proj_b):
    grid_spec = pltpu.PrefetchScalarGridSpec(
        num_scalar_prefetch=1,
        grid=(NBLK,),
        in_specs=[
            pl.BlockSpec((BLK, N_EMBD), lambda g, be: (g, 0)),
            pl.BlockSpec((1, N_EMBD, D_FF), lambda g, be: (be[0, g], 0, 0)),
            pl.BlockSpec((1, 1, D_FF), lambda g, be: (be[0, g], 0, 0)),
            pl.BlockSpec((1, D_FF, N_EMBD), lambda g, be: (be[0, g], 0, 0)),
            pl.BlockSpec((1, 1, N_EMBD), lambda g, be: (be[0, g], 0, 0)),
        ],
        out_specs=pl.BlockSpec((BLK, N_EMBD), lambda g, be: (g, 0)),
    )
    return pl.pallas_call(
        _gffn_body,
        grid_spec=grid_spec,
        out_shape=jax.ShapeDtypeStruct((PADN, N_EMBD), jnp.float32),
    )(be, perm_x,
      fc_w.astype(jnp.bfloat16),
      fc_b.reshape(NUM_EXPERTS, 1, D_FF),
      proj_w.astype(jnp.bfloat16),
      proj_b.reshape(NUM_EXPERTS, 1, N_EMBD))


_CTOK = 16


_N_CCH = 256 // _CTOK


def _sc_combine_body(yw_hbm, pos_hbm, gw0_hbm, gw1_hbm, out_hbm,
                     idx_v, gw0_v, gw1_v, g0a, g1a, g0b, g1b, st,
                     semga, semgb, sems):
    c = lax.axis_index("c")
    s = lax.axis_index("s")
    w = 2 * s + c
    tok_base = _TOK_PER_W // 2 * w

    pairs = ((g0a, g1a, semga), (g0b, g1b, semgb))

    pltpu.sync_copy(pos_hbm.at[pl.ds(8 * w, 8)], idx_v.at[pl.ds(0, 8)])
    pltpu.sync_copy(pos_hbm.at[pl.ds(256 + 8 * w, 8)], idx_v.at[pl.ds(8, 8)])
    pltpu.sync_copy(gw0_hbm.at[pl.ds(tok_base, 256)], gw0_v.at[pl.ds(0, 256)])
    pltpu.sync_copy(gw1_hbm.at[pl.ds(tok_base, 256)], gw1_v.at[pl.ds(0, 256)])

    def idx0(ch):
        return idx_v.at[ch // 2, pl.ds((ch % 2) * 16, 16)]

    def idx1(ch):
        return idx_v.at[8 + ch // 2, pl.ds((ch % 2) * 16, 16)]

    def start_gather(ch):
        g0, g1, sg = pairs[ch % 2]
        pltpu.async_copy(yw_hbm.at[idx0(ch)], g0, sg)
        pltpu.async_copy(yw_hbm.at[idx1(ch)], g1, sg)

    start_gather(0)
    start_gather(1)
    for ch in range(_N_CCH):
        g0, g1, sg = pairs[ch % 2]
        pltpu.make_async_copy(yw_hbm.at[idx0(ch)], g0, sg).wait()
        pltpu.make_async_copy(yw_hbm.at[idx1(ch)], g1, sg).wait()
        if ch >= 1:
            tp = tok_base + _CTOK * (ch - 1)
            pltpu.make_async_copy(st, out_hbm.at[pl.ds(tp, _CTOK)],
                                  sems).wait()

        def add_row(r, _):
            gv0 = jnp.full((16,), gw0_v[pl.ds(_CTOK * ch + r, 16)][0],
                           jnp.float32)
            gv1 = jnp.full((16,), gw1_v[pl.ds(_CTOK * ch + r, 16)][0],
                           jnp.float32)
            for cc in range(N_EMBD // 16):
                st[r, pl.ds(cc * 16, 16)] = (g0[r, pl.ds(cc * 16, 16)] * gv0
                                             + g1[r, pl.ds(cc * 16, 16)] * gv1)
            return 0

        lax.fori_loop(0, _CTOK, add_row, 0)
        t0 = tok_base + _CTOK * ch
        pltpu.async_copy(st, out_hbm.at[pl.ds(t0, _CTOK)], sems)
        if ch + 2 < _N_CCH:
            start_gather(ch + 2)
    t0 = tok_base + _CTOK * (_N_CCH - 1)
    pltpu.make_async_copy(st, out_hbm.at[pl.ds(t0, _CTOK)], sems).wait()


def _run_sc_combine(yw, pos, gw0, gw1):
    mesh = plsc.VectorSubcoreMesh(core_axis_name="c", subcore_axis_name="s")
    buf = pltpu.VMEM((_CTOK, N_EMBD), jnp.float32)
    f = pl.kernel(
        _sc_combine_body,
        out_type=jax.ShapeDtypeStruct((N_TOK, N_EMBD), jnp.float32),
        mesh=mesh,
        scratch_types=[
            pltpu.VMEM((16, 32), jnp.int32),
            pltpu.VMEM((272,), jnp.float32),
            pltpu.VMEM((272,), jnp.float32),
            buf, buf, buf, buf, buf,
            pltpu.SemaphoreType.DMA,
            pltpu.SemaphoreType.DMA,
            pltpu.SemaphoreType.DMA,
        ],
        compiler_params=pltpu.CompilerParams(needs_layout_passes=False),
    )
    return f(yw, pos, gw0.reshape(-1), gw1.reshape(-1))


def kernel(x, city, delta_t_info, delta_dis_info, delta_rg_info,
           delta_entropy_info, city_embeddings, router_w, router_b,
           fc_w, fc_b, proj_w, proj_b):
    b, t, d = x.shape
    n = b * t
    x2d = x.reshape(n, d)
    d1 = delta_t_info.reshape(n, -1)
    d2 = delta_dis_info.reshape(n, -1)
    d3 = delta_rg_info.reshape(n, -1)
    d4 = delta_entropy_info.reshape(n, -1)

    ce = city_embeddings[city]
    w_ce = jax.lax.dynamic_slice_in_dim(router_w, N_EMBD, CITY_DIM, 0)
    bias_full = (jnp.dot(ce, w_ce) + router_b).reshape(1, NUM_EXPERTS)

    gate1, i1, i2, w1, w2, cnt = _run_router(x2d, d1, d2, d3, d4, router_w,
                                             bias_full)
    ex0 = i1.reshape(-1)
    ex1 = i2.reshape(-1)

    pos, be_act, perm_x = _run_sc_route(ex0, ex1, x2d, cnt)
    yw = _run_grouped_ffn(be_act, perm_x, fc_w, fc_b, proj_w, proj_b)
    out2d = _run_sc_combine(yw, pos, w1, w2)
    return out2d.reshape(b, t, d), gate1.reshape(b, t, NUM_EXPERTS)


# speedup vs baseline: 1.0622x; 1.0375x over previous
import functools

import jax
import jax.numpy as jnp
from jax import lax
from jax.experimental import pallas as pl
from jax.experimental.pallas import tpu as pltpu
from jax.experimental.pallas import tpu_sc as plsc

B, T, N_EMBD = 4, 2048, 768
NUM_EXPERTS, TOP_K = 8, 2
CITY_DIM = 32
D_FF = 4 * N_EMBD
N_TOK = B * T
N_SLOT = N_TOK * TOP_K

BLK = 512
NBLK = 40
PADN = NBLK * BLK

_NC, _NS = 2, 16
_NW = _NC * _NS
_SLOT_PER_W = N_SLOT // _NW
_TOK_PER_W = _SLOT_PER_W

_SQRT_2_OVER_PI = 0.7978845608028654


def _gelu_tanh(x):
    return 0.5 * x * (1.0 + jnp.tanh(_SQRT_2_OVER_PI * (x + 0.044715 * x * x * x)))


def _router_body(x_ref, d1_ref, d2_ref, d3_ref, d4_ref,
                 wx_ref, w1_ref, w2_ref, w3_ref, w4_ref, bias_ref,
                 gate1_ref, i1_ref, i2_ref, w1o_ref, w2o_ref, cnt_ref):
    logits = jnp.dot(x_ref[...], wx_ref[...], preferred_element_type=jnp.float32)
    logits += jnp.dot(d1_ref[...], w1_ref[...], preferred_element_type=jnp.float32)
    logits += jnp.dot(d2_ref[...], w2_ref[...], preferred_element_type=jnp.float32)
    logits += jnp.dot(d3_ref[...], w3_ref[...], preferred_element_type=jnp.float32)
    logits += jnp.dot(d4_ref[...], w4_ref[...], preferred_element_type=jnp.float32)
    logits += bias_ref[...]

    m1 = jnp.max(logits, axis=-1, keepdims=True)
    e = jnp.exp(logits - m1)
    gate1_ref[...] = e / jnp.sum(e, axis=-1, keepdims=True)

    iota = jax.lax.broadcasted_iota(jnp.int32, logits.shape, 1)
    i1 = jnp.min(jnp.where(logits == m1, iota, NUM_EXPERTS), axis=-1,
                 keepdims=True)
    masked = jnp.where(iota == i1, -jnp.inf, logits)
    m2 = jnp.max(masked, axis=-1, keepdims=True)
    i2 = jnp.min(jnp.where(masked == m2, iota, NUM_EXPERTS), axis=-1,
                 keepdims=True)
    w_top1 = 1.0 / (1.0 + jnp.exp(m2 - m1))
    i1_ref[...] = i1
    i2_ref[...] = i2
    w1o_ref[...] = w_top1
    w2o_ref[...] = 1.0 - w_top1

    lane16 = jax.lax.broadcasted_iota(jnp.int32, (512, 16), 1)
    rows = []
    for src_col, _k in ((i1, 0), (i2, 1)):
        for h in range(2):
            sl = src_col[h * 512:(h + 1) * 512]
            oh = (sl == lane16).astype(jnp.int32)
            rows.append(jnp.sum(oh, axis=0, keepdims=True))
    cnt_ref[...] = jnp.concatenate(rows, axis=0)[None]


def _run_router(x2d, d1, d2, d3, d4, router_w, bias_full):
    blk = 1024
    grid = (N_TOK // blk,)
    wx = router_w[:N_EMBD]
    o = N_EMBD + CITY_DIM
    w1 = router_w[o:o + 192]
    w2 = router_w[o + 192:o + 384]
    w3 = router_w[o + 384:o + 480]
    w4 = router_w[o + 480:o + 576]

    def tok_block(i):
        return (i, 0)

    def full(i):
        return (0, 0)

    col = pl.BlockSpec((blk, 1), tok_block)
    return pl.pallas_call(
        _router_body,
        grid=grid,
        in_specs=[
            pl.BlockSpec((blk, N_EMBD), tok_block),
            pl.BlockSpec((blk, 192), tok_block),
            pl.BlockSpec((blk, 192), tok_block),
            pl.BlockSpec((blk, 96), tok_block),
            pl.BlockSpec((blk, 96), tok_block),
            pl.BlockSpec((N_EMBD, NUM_EXPERTS), full),
            pl.BlockSpec((192, NUM_EXPERTS), full),
            pl.BlockSpec((192, NUM_EXPERTS), full),
            pl.BlockSpec((96, NUM_EXPERTS), full),
            pl.BlockSpec((96, NUM_EXPERTS), full),
            pl.BlockSpec((1, NUM_EXPERTS), full),
        ],
        out_specs=[
            pl.BlockSpec((blk, NUM_EXPERTS), tok_block),
            col, col, col, col,
            pl.BlockSpec((1, 4, 16), lambda i: (i, 0, 0)),
        ],
        out_shape=[
            jax.ShapeDtypeStruct((N_TOK, NUM_EXPERTS), jnp.float32),
            jax.ShapeDtypeStruct((N_TOK, 1), jnp.int32),
            jax.ShapeDtypeStruct((N_TOK, 1), jnp.int32),
            jax.ShapeDtypeStruct((N_TOK, 1), jnp.float32),
            jax.ShapeDtypeStruct((N_TOK, 1), jnp.float32),
            jax.ShapeDtypeStruct((_NW // 4, 4, 16), jnp.int32),
        ],
    )(x2d, d1, d2, d3, d4, wx, w1, w2, w3, w4, bias_full)


_GRP = 16
_CHUNK = 128
_N_CHUNK = _SLOT_PER_W // _CHUNK
_POS_GRPS = _SLOT_PER_W // _GRP


def _sc_route_body(ex0_hbm, ex1_hbm, x_hbm, cnt_hbm,
                   pos_hbm, be_hbm, px_hbm,
                   ex_v, all_v, pos_v, be_v, xbuf, xbuf2, xbuf3, xbuf4,
                   seml0, seml1, seml2, seml3, sems0, sems1, sems2, sems3):
    c = lax.axis_index("c")
    s = lax.axis_index("s")
    w = 2 * s + c
    lane = lax.iota(jnp.int32, 16)
    zero16 = jnp.zeros((16,), jnp.int32)

    pltpu.sync_copy(cnt_hbm, all_v)
    row_w = 4 * ((w % 16) // 2) + 2 * (w // 16) + (w % 16) % 2
    tot = zero16
    pre = zero16
    for r in range(_NW):
        row = all_v[r // 4, r % 4, :]
        tot = tot + row
        pre = pre + row * (r < row_w).astype(jnp.int32)
    padded = ((tot + (BLK - 1)) // BLK) * BLK
    base_excl = plsc.cumsum(padded) - padded
    start = base_excl + pre

    @pl.when(w < 16)
    def _():
        pltpu.sync_copy(ex0_hbm.at[pl.ds(512 * w, 512)],
                        ex_v.at[pl.ds(0, 512)])

    @pl.when(w >= 16)
    def _():
        pltpu.sync_copy(ex1_hbm.at[pl.ds(512 * (w - 16), 512)],
                        ex_v.at[pl.ds(0, 512)])

    def pos_grp(g, start_vec):
        v = ex_v[pl.ds(g * 16, 16)]
        posv = jnp.zeros((16,), jnp.int32)
        upd = start_vec
        for e in range(NUM_EXPERTS):
            mi = (v == e).astype(jnp.int32)
            csum = plsc.cumsum(mi)
            start_e = jnp.sum(jnp.where(lane == e, start_vec, 0))
            posv = posv + mi * (start_e + csum - 1)
            cnt_e = jnp.sum(mi)
            upd = upd + jnp.where(lane == e, cnt_e, 0)
        pos_v[g // 2, pl.ds((g % 2) * 16, 16)] = posv
        return upd

    lax.fori_loop(0, _POS_GRPS, pos_grp, start)

    pltpu.sync_copy(pos_v, pos_hbm.at[pl.ds(16 * w, 16)])

    @pl.when(w == 0)
    def _():
        base_blk = base_excl // BLK
        tot_blk = jnp.sum(padded // BLK)
        sb = [jnp.sum(jnp.where(lane == e, base_blk, 0))
              for e in range(NUM_EXPERTS)]
        for j in range(128 // 16):
            blkid = lane + 16 * j
            bev = jnp.zeros((16,), jnp.int32)
            for e in range(NUM_EXPERTS):
                bev = bev + (blkid >= sb[e]).astype(jnp.int32)
            be_v[0, pl.ds(16 * j, 16)] = bev - 1
            be_v[1, pl.ds(16 * j, 16)] = (blkid < tot_blk).astype(jnp.int32)
        pltpu.sync_copy(be_v, be_hbm)

    tok_base = 512 * (w % 16)
    xbs = (xbuf, xbuf2, xbuf3, xbuf4)
    sls = (seml0, seml1, seml2, seml3)
    sss = (sems0, sems1, sems2, sems3)

    def ld(ch):
        pltpu.async_copy(x_hbm.at[pl.ds(tok_base + 32 * ch, 32)],
                         xbs[ch % 4], sls[ch % 4])

    def idx_of(ch):
        return pos_v.at[ch]

    ld(0)
    for ch in range(16):
        p = ch % 4
        if ch + 1 < 16:
            q = (ch + 1) % 4
            if ch + 1 >= 4:
                pltpu.make_async_copy(xbs[q], px_hbm.at[idx_of(ch - 3)],
                                      sss[q]).wait()
            ld(ch + 1)
        pltpu.make_async_copy(x_hbm.at[pl.ds(tok_base + 32 * ch, 32)],
                              xbs[p], sls[p]).wait()
        pltpu.async_copy(xbs[p], px_hbm.at[idx_of(ch)], sss[p])
    for ch in range(12, 16):
        p = ch % 4
        pltpu.make_async_copy(xbs[p], px_hbm.at[idx_of(ch)], sss[p]).wait()


def _run_sc_route(ex0, ex1, x2d, cnt):
    mesh = plsc.VectorSubcoreMesh(core_axis_name="c", subcore_axis_name="s")
    f = pl.kernel(
        _sc_route_body,
        out_type=[
            jax.ShapeDtypeStruct((N_SLOT // 32, 32), jnp.int32),
            jax.ShapeDtypeStruct((2, 128), jnp.int32),
            jax.ShapeDtypeStruct((PADN, N_EMBD), jnp.float32),
        ],
        mesh=mesh,
        scratch_types=[
            pltpu.VMEM((1024,), jnp.int32),
            pltpu.VMEM((_NW // 4, 4, 16), jnp.int32),
            pltpu.VMEM((16, 32), jnp.int32),
            pltpu.VMEM((2, 128), jnp.int32),
            pltpu.VMEM((32, N_EMBD), jnp.float32),
            pltpu.VMEM((32, N_EMBD), jnp.float32),
            pltpu.VMEM((32, N_EMBD), jnp.float32),
            pltpu.VMEM((32, N_EMBD), jnp.float32),
            pltpu.SemaphoreType.DMA,
            pltpu.SemaphoreType.DMA,
            pltpu.SemaphoreType.DMA,
            pltpu.SemaphoreType.DMA,
            pltpu.SemaphoreType.DMA,
            pltpu.SemaphoreType.DMA,
            pltpu.SemaphoreType.DMA,
            pltpu.SemaphoreType.DMA,
        ],
        compiler_params=pltpu.CompilerParams(needs_layout_passes=False),
    )
    return f(ex0, ex1, x2d, cnt)


def _gffn_body(be_ref, x_ref, fcw_ref, fcb_ref, pjw_ref, pjb_ref,
               out_ref):
    g = pl.program_id(0)

    @pl.when(be_ref[1, g] == 1)
    def _():
        xb = x_ref[...].astype(jnp.bfloat16)
        hid = jnp.dot(xb, fcw_ref[0], preferred_element_type=jnp.float32)
        hid = _gelu_tanh(hid + fcb_ref[0])
        out = jnp.dot(hid.astype(jnp.bfloat16), pjw_ref[0],
                      preferred_element_type=jnp.float32)
        out_ref[...] = out + pjb_ref[0]


def _run_grouped_ffn(be, perm_x, fc_w, fc_b, proj_w, proj_b):
    grid_spec = pltpu.PrefetchScalarGridSpec(
        num_scalar_prefetch=1,
        grid=(NBLK,),
        in_specs=[
            pl.BlockSpec((BLK, N_EMBD), lambda g, be: (g, 0)),
            pl.BlockSpec((1, N_EMBD, D_FF), lambda g, be: (be[0, g], 0, 0)),
            pl.BlockSpec((1, 1, D_FF), lambda g, be: (be[0, g], 0, 0)),
            pl.BlockSpec((1, D_FF, N_EMBD), lambda g, be: (be[0, g], 0, 0)),
            pl.BlockSpec((1, 1, N_EMBD), lambda g, be: (be[0, g], 0, 0)),
        ],
        out_specs=pl.BlockSpec((BLK, N_EMBD), lambda g, be: (g, 0)),
    )
    return pl.pallas_call(
        _gffn_body,
        grid_spec=grid_spec,
        out_shape=jax.ShapeDtypeStruct((PADN, N_EMBD), jnp.float32),
    )(be, perm_x,
      fc_w.astype(jnp.bfloat16),
      fc_b.reshape(NUM_EXPERTS, 1, D_FF),
      proj_w.astype(jnp.bfloat16),
      proj_b.reshape(NUM_EXPERTS, 1, N_EMBD))


_CTOK = 16


_N_CCH = 256 // _CTOK


def _sc_combine_body(yw_hbm, pos_hbm, gw0_hbm, gw1_hbm, out_hbm,
                     idx_v, gw0_v, gw1_v, g0a, g1a, g0b, g1b, st,
                     semga, semgb, sems):
    c = lax.axis_index("c")
    s = lax.axis_index("s")
    w = 2 * s + c
    tok_base = _TOK_PER_W // 2 * w

    pairs = ((g0a, g1a, semga), (g0b, g1b, semgb))

    pltpu.sync_copy(pos_hbm.at[pl.ds(8 * w, 8)], idx_v.at[pl.ds(0, 8)])
    pltpu.sync_copy(pos_hbm.at[pl.ds(256 + 8 * w, 8)], idx_v.at[pl.ds(8, 8)])
    pltpu.sync_copy(gw0_hbm.at[pl.ds(tok_base, 256)], gw0_v.at[pl.ds(0, 256)])
    pltpu.sync_copy(gw1_hbm.at[pl.ds(tok_base, 256)], gw1_v.at[pl.ds(0, 256)])

    def idx0(ch):
        return idx_v.at[ch // 2, pl.ds((ch % 2) * 16, 16)]

    def idx1(ch):
        return idx_v.at[8 + ch // 2, pl.ds((ch % 2) * 16, 16)]

    def start_gather(ch):
        g0, g1, sg = pairs[ch % 2]
        pltpu.async_copy(yw_hbm.at[idx0(ch)], g0, sg)
        pltpu.async_copy(yw_hbm.at[idx1(ch)], g1, sg)

    start_gather(0)
    start_gather(1)
    for ch in range(_N_CCH):
        g0, g1, sg = pairs[ch % 2]
        pltpu.make_async_copy(yw_hbm.at[idx0(ch)], g0, sg).wait()
        pltpu.make_async_copy(yw_hbm.at[idx1(ch)], g1, sg).wait()
        if ch >= 1:
            tp = tok_base + _CTOK * (ch - 1)
            pltpu.make_async_copy(st, out_hbm.at[pl.ds(tp, _CTOK)],
                                  sems).wait()

        def add_row(r, _):
            gv0 = jnp.full((16,), gw0_v[pl.ds(_CTOK * ch + r, 16)][0],
                           jnp.float32)
            gv1 = jnp.full((16,), gw1_v[pl.ds(_CTOK * ch + r, 16)][0],
                           jnp.float32)
            for cc in range(N_EMBD // 16):
                st[r, pl.ds(cc * 16, 16)] = (g0[r, pl.ds(cc * 16, 16)] * gv0
                                             + g1[r, pl.ds(cc * 16, 16)] * gv1)
            return 0

        lax.fori_loop(0, _CTOK, add_row, 0)
        t0 = tok_base + _CTOK * ch
        pltpu.async_copy(st, out_hbm.at[pl.ds(t0, _CTOK)], sems)
        if ch + 2 < _N_CCH:
            start_gather(ch + 2)
    t0 = tok_base + _CTOK * (_N_CCH - 1)
    pltpu.make_async_copy(st, out_hbm.at[pl.ds(t0, _CTOK)], sems).wait()


def _run_sc_combine(yw, pos, gw0, gw1):
    mesh = plsc.VectorSubcoreMesh(core_axis_name="c", subcore_axis_name="s")
    buf = pltpu.VMEM((_CTOK, N_EMBD), jnp.float32)
    f = pl.kernel(
        _sc_combine_body,
        out_type=jax.ShapeDtypeStruct((N_TOK, N_EMBD), jnp.float32),
        mesh=mesh,
        scratch_types=[
            pltpu.VMEM((16, 32), jnp.int32),
            pltpu.VMEM((272,), jnp.float32),
            pltpu.VMEM((272,), jnp.float32),
            buf, buf, buf, buf, buf,
            pltpu.SemaphoreType.DMA,
            pltpu.SemaphoreType.DMA,
            pltpu.SemaphoreType.DMA,
        ],
        compiler_params=pltpu.CompilerParams(needs_layout_passes=False),
    )
    return f(yw, pos, gw0.reshape(-1), gw1.reshape(-1))


def kernel(x, city, delta_t_info, delta_dis_info, delta_rg_info,
           delta_entropy_info, city_embeddings, router_w, router_b,
           fc_w, fc_b, proj_w, proj_b):
    b, t, d = x.shape
    n = b * t
    x2d = x.reshape(n, d)
    d1 = delta_t_info.reshape(n, -1)
    d2 = delta_dis_info.reshape(n, -1)
    d3 = delta_rg_info.reshape(n, -1)
    d4 = delta_entropy_info.reshape(n, -1)

    ce = city_embeddings[city]
    w_ce = jax.lax.dynamic_slice_in_dim(router_w, N_EMBD, CITY_DIM, 0)
    bias_full = (jnp.dot(ce, w_ce) + router_b).reshape(1, NUM_EXPERTS)

    gate1, i1, i2, w1, w2, cnt = _run_router(x2d, d1, d2, d3, d4, router_w,
                                             bias_full)
    ex0 = i1.reshape(-1)
    ex1 = i2.reshape(-1)

    pos, be_act, perm_x = _run_sc_route(ex0, ex1, x2d, cnt)
    yw = _run_grouped_ffn(be_act, perm_x, fc_w, fc_b, proj_w, proj_b)
    out2d = _run_sc_combine(yw, pos, w1, w2)
    return out2d.reshape(b, t, d), gate1.reshape(b, t, NUM_EXPERTS)
